# baseline (speedup 1.0000x reference)
import jax
import jax.numpy as jnp
from jax.experimental import pallas as pl
from jax.experimental.pallas import tpu as pltpu

B = 512
P = 64
K = 2048
G = 64
W = 128


def _srp_tc_kernel(tau0_ref, front_ref, back_ref, out_ref, oh_ref):
    Bb = front_ref.shape[0]

    @pl.when(pl.program_id(0) == 0)
    def _build_onehot():
        idx = tau0_ref[...]
        idx = jnp.where(idx < 0, idx + 2 * W, idx)
        iota = jax.lax.broadcasted_iota(jnp.int32, (2 * W, G), 0)
        for p in range(P):
            oh_ref[p, :, :] = (iota == idx[p : p + 1, :]).astype(jnp.float32)

    acc = jnp.zeros((Bb, G), dtype=jnp.float32)
    for p in range(P):
        xw = jnp.concatenate([front_ref[:, p, :], back_ref[:, p, :]], axis=-1)
        acc += jnp.dot(xw, oh_ref[p, :, :], preferred_element_type=jnp.float32)
    maps = acc + 1e-12
    out_ref[...] = maps / jnp.max(maps, axis=-1, keepdims=True)


@jax.jit
def kernel(x, tau0):
    xr = x.reshape(B, P, K)
    t0 = tau0.reshape(P, G)

    Bb = 128
    grid = (B // Bb,)
    return pl.pallas_call(
        _srp_tc_kernel,
        grid=grid,
        in_specs=[
            pl.BlockSpec((P, G), lambda i: (0, 0)),
            pl.BlockSpec((Bb, P, W), lambda i: (i, 0, 0)),
            pl.BlockSpec((Bb, P, W), lambda i: (i, 0, K // W - 1)),
        ],
        out_specs=pl.BlockSpec((Bb, G), lambda i: (i, 0)),
        out_shape=jax.ShapeDtypeStruct((B, G), jnp.float32),
        scratch_shapes=[pltpu.VMEM((P, 2 * W, G), jnp.float32)],
        compiler_params=pltpu.CompilerParams(
            dimension_semantics=("arbitrary",),
        ),
    )(t0, xr, xr)


# speedup vs baseline: 1.4226x; 1.4226x over previous
import jax
import jax.numpy as jnp
from jax.experimental import pallas as pl
from jax.experimental.pallas import tpu as pltpu

B = 512
P = 64
K = 2048
G = 64
W = 128
V = 16


def _srp_tc_kernel(tau0_ref, front_ref, back_ref, out_ref, ohf_ref, ohb_ref):
    Bb = front_ref.shape[0]

    @pl.when(pl.program_id(0) == 0)
    def _build_onehot():
        t = tau0_ref[...]
        iota = jax.lax.broadcasted_iota(jnp.int32, (V, G), 0)
        for p in range(P):
            q, s = divmod(p, 8)
            tp = t[p : p + 1, :]
            ohf_ref[q, pl.ds(s * V, V), :] = (iota == tp).astype(jnp.float32)
            ohb_ref[q, pl.ds(s * V, V), :] = (iota == tp + V).astype(jnp.float32)

    acc = jnp.zeros((Bb, G), dtype=jnp.float32)
    for q in range(P // 8):
        xf = front_ref[:, pl.ds(8 * q, 8), pl.ds(0, V)].reshape(Bb, 8 * V)
        xb = back_ref[:, pl.ds(8 * q, 8), pl.ds(W - V, V)].reshape(Bb, 8 * V)
        acc += jnp.dot(xf, ohf_ref[q], preferred_element_type=jnp.float32)
        acc += jnp.dot(xb, ohb_ref[q], preferred_element_type=jnp.float32)
    maps = acc + 1e-12
    out_ref[...] = maps / jnp.max(maps, axis=-1, keepdims=True)


@jax.jit
def kernel(x, tau0):
    xr = x.reshape(B, P, K)
    t0 = tau0.reshape(P, G)

    Bb = 64
    grid = (B // Bb,)
    return pl.pallas_call(
        _srp_tc_kernel,
        grid=grid,
        in_specs=[
            pl.BlockSpec((P, G), lambda i: (0, 0)),
            pl.BlockSpec((Bb, P, W), lambda i: (i, 0, 0)),
            pl.BlockSpec((Bb, P, W), lambda i: (i, 0, K // W - 1)),
        ],
        out_specs=pl.BlockSpec((Bb, G), lambda i: (i, 0)),
        out_shape=jax.ShapeDtypeStruct((B, G), jnp.float32),
        scratch_shapes=[
            pltpu.VMEM((P // 8, 8 * V, G), jnp.float32),
            pltpu.VMEM((P // 8, 8 * V, G), jnp.float32),
        ],
        compiler_params=pltpu.CompilerParams(
            dimension_semantics=("arbitrary",),
        ),
    )(t0, xr, xr)
